# Initial kernel scaffold; baseline (speedup 1.0000x reference)
import jax
import jax.numpy as jnp
from jax.experimental import pallas as pl


def kernel(pos_id, pe):
    raise NotImplementedError("write your pallas kernel here")


# speedup vs baseline: 2.1860x; 2.1860x over previous
import functools

import jax
import jax.numpy as jnp
from jax import lax
from jax.experimental import pallas as pl
from jax.experimental.pallas import tpu as pltpu
from jax.experimental.pallas import tpu_sc as plsc

WIDTH = 1024
NUM_CORES = 2
NUM_SUBCORES = 16
NW = NUM_CORES * NUM_SUBCORES
CHUNK = 64


@functools.partial(jax.jit, static_argnames=("total",))
def _gather(idx_flat, pe, total):
    b_per_w = total // NW
    n_chunks = b_per_w // CHUNK
    mesh = plsc.VectorSubcoreMesh(core_axis_name="c", subcore_axis_name="s")

    @functools.partial(
        pl.kernel,
        mesh=mesh,
        out_type=jax.ShapeDtypeStruct((total, WIDTH), jnp.float32),
        scratch_types=[
            pltpu.VMEM((b_per_w,), jnp.int32),
            pltpu.VMEM((CHUNK, WIDTH), jnp.float32),
            pltpu.SemaphoreType.DMA,
        ],
    )
    def k(idx_hbm, table_hbm, out_hbm, idx_v, rows_v, sem):
        wid = lax.axis_index("s") * NUM_CORES + lax.axis_index("c")
        base = wid * b_per_w
        pltpu.sync_copy(idx_hbm.at[pl.ds(base, b_per_w)], idx_v)

        def body(ch, carry):
            off = ch * CHUNK
            pltpu.async_copy(
                table_hbm.at[idx_v.at[pl.ds(off, CHUNK)]], rows_v, sem
            ).wait()
            pltpu.sync_copy(rows_v, out_hbm.at[pl.ds(base + off, CHUNK)])
            return carry

        lax.fori_loop(0, n_chunks, body, 0)

    return k(idx_flat, pe)


def kernel(pos_id, pe):
    b, s = pos_id.shape
    total = b * s
    out = _gather(pos_id.reshape(total), pe, total)
    return out.reshape(b, s, WIDTH)


# speedup vs baseline: 2.3706x; 1.0844x over previous
import functools

import jax
import jax.numpy as jnp
from jax import lax
from jax.experimental import pallas as pl
from jax.experimental.pallas import tpu as pltpu
from jax.experimental.pallas import tpu_sc as plsc

WIDTH = 1024
NUM_CORES = 2
NUM_SUBCORES = 16
NW = NUM_CORES * NUM_SUBCORES
CHUNK = 32


@functools.partial(jax.jit, static_argnames=("total",))
def _gather(idx_flat, pe, total):
    b_per_w = total // NW
    n_chunks = b_per_w // CHUNK
    mesh = plsc.VectorSubcoreMesh(core_axis_name="c", subcore_axis_name="s")

    @functools.partial(
        pl.kernel,
        mesh=mesh,
        out_type=jax.ShapeDtypeStruct((total, WIDTH), jnp.float32),
        scratch_types=[
            pltpu.VMEM((b_per_w,), jnp.int32),
            pltpu.VMEM((CHUNK, WIDTH), jnp.float32),
            pltpu.VMEM((CHUNK, WIDTH), jnp.float32),
            pltpu.SemaphoreType.DMA,
            pltpu.SemaphoreType.DMA,
            pltpu.SemaphoreType.DMA,
            pltpu.SemaphoreType.DMA,
        ],
    )
    def k(idx_hbm, table_hbm, out_hbm, idx_v, buf0, buf1,
          gsem0, gsem1, ssem0, ssem1):
        wid = lax.axis_index("s") * NUM_CORES + lax.axis_index("c")
        base = wid * b_per_w
        pltpu.sync_copy(idx_hbm.at[pl.ds(base, b_per_w)], idx_v)

        bufs = (buf0, buf1)
        gsems = (gsem0, gsem1)
        ssems = (ssem0, ssem1)

        def gather(g, b):
            off = g * CHUNK
            return pltpu.make_async_copy(
                table_hbm.at[idx_v.at[pl.ds(off, CHUNK)]], bufs[b], gsems[b])

        def store(g, b):
            off = g * CHUNK
            return pltpu.make_async_copy(
                bufs[b], out_hbm.at[pl.ds(base + off, CHUNK)], ssems[b])

        gather(0, 0).start()
        gather(1, 1).start()
        gather(0, 0).wait()
        store(0, 0).start()

        def pair(p, carry):
            g0 = 2 * p + 1
            g1 = g0 + 1
            store(g0 - 1, 0).wait()
            gather(g0 + 1, 0).start()
            gather(g0, 1).wait()
            store(g0, 1).start()
            store(g1 - 1, 1).wait()
            gather(g1 + 1, 1).start()
            gather(g1, 0).wait()
            store(g1, 0).start()
            return carry

        lax.fori_loop(0, (n_chunks - 2) // 2, pair, 0)

        g_last = n_chunks - 1
        store(g_last - 1, 0).wait()
        gather(g_last, 1).wait()
        store(g_last, 1).start()
        store(g_last, 1).wait()

    return k(idx_flat, pe)


def kernel(pos_id, pe):
    b, s = pos_id.shape
    total = b * s
    out = _gather(pos_id.reshape(total), pe, total)
    return out.reshape(b, s, WIDTH)


# speedup vs baseline: 2.3795x; 1.0037x over previous
import functools

import jax
import jax.numpy as jnp
from jax import lax
from jax.experimental import pallas as pl
from jax.experimental.pallas import tpu as pltpu
from jax.experimental.pallas import tpu_sc as plsc

WIDTH = 1024
NUM_CORES = 2
NUM_SUBCORES = 16
NW = NUM_CORES * NUM_SUBCORES
CHUNK = 16
NBUF = 4


@functools.partial(jax.jit, static_argnames=("total",))
def _gather(idx_flat, pe, total):
    b_per_w = total // NW
    n_chunks = b_per_w // CHUNK
    mesh = plsc.VectorSubcoreMesh(core_axis_name="c", subcore_axis_name="s")

    @functools.partial(
        pl.kernel,
        mesh=mesh,
        out_type=jax.ShapeDtypeStruct((total, WIDTH), jnp.float32),
        scratch_types=(
            [pltpu.VMEM((b_per_w,), jnp.int32)]
            + [pltpu.VMEM((CHUNK, WIDTH), jnp.float32)] * NBUF
            + [pltpu.SemaphoreType.DMA] * (2 * NBUF)
        ),
    )
    def k(idx_hbm, table_hbm, out_hbm, idx_v, *bufs_sems):
        bufs = bufs_sems[:NBUF]
        gsems = bufs_sems[NBUF:2 * NBUF]
        ssems = bufs_sems[2 * NBUF:]
        wid = lax.axis_index("s") * NUM_CORES + lax.axis_index("c")
        base = wid * b_per_w
        pltpu.sync_copy(idx_hbm.at[pl.ds(base, b_per_w)], idx_v)

        def gather(g, b):
            off = g * CHUNK
            return pltpu.make_async_copy(
                table_hbm.at[idx_v.at[pl.ds(off, CHUNK)]], bufs[b], gsems[b])

        def store(g, b):
            off = g * CHUNK
            return pltpu.make_async_copy(
                bufs[b], out_hbm.at[pl.ds(base + off, CHUNK)], ssems[b])

        gather(0, 0).start()
        gather(1, 1).start()
        gather(0, 0).wait()
        store(0, 0).start()
        gather(2, 2).start()
        gather(1, 1).wait()
        store(1, 1).start()
        gather(3, 3).start()

        def quad(q, carry):
            g_base = 4 * q + 2
            for j in range(4):
                g = g_base + j
                p = (2 + j) % NBUF
                store(g - 2, (p + 2) % NBUF).wait()
                gather(g, p).wait()
                store(g, p).start()
                gather(g + 2, (p + 2) % NBUF).start()
            return carry

        lax.fori_loop(0, (n_chunks - 4) // 4, quad, 0)

        g = n_chunks - 2
        store(g - 2, 0).wait()
        gather(g, 2).wait()
        store(g, 2).start()
        g = n_chunks - 1
        store(g - 2, 1).wait()
        gather(g, 3).wait()
        store(g, 3).start()
        store(n_chunks - 2, 2).wait()
        store(n_chunks - 1, 3).wait()

    return k(idx_flat, pe)


def kernel(pos_id, pe):
    b, s = pos_id.shape
    total = b * s
    out = _gather(pos_id.reshape(total), pe, total)
    return out.reshape(b, s, WIDTH)
